# baseline (speedup 1.0000x reference)
import jax
import jax.numpy as jnp
from jax import lax
from jax.experimental import pallas as pl
from jax.experimental.pallas import tpu as pltpu

_EPS2 = 1.6 * 1.6
_MINS = 5
_N = 4096
_D = 8
_MAXIT = 64
_SENT = float(_N)
_RB = 256
_NB = _N // _RB


def _row(x):
    return x.reshape(1, -1)


def _dbscan_body(x_ref, out_ref, m_ref, deg_ref, vec_ref, hook_ref,
                 labm_ref, prev_ref, part_ref):
    x = x_ref[...]
    sq = jnp.sum(x * x, axis=1)
    sq_row = _row(sq)
    iota_row = lax.broadcasted_iota(jnp.int32, (1, _N), 1).astype(jnp.float32)

    hook_ref[...] = sq_row

    def build_blk(b, _):
        xb = x_ref[pl.ds(b * _RB, _RB), :]
        p = lax.dot_general(xb, x, (((1,), (1,)), ((), ())),
                            preferred_element_type=jnp.float32)
        sqb = hook_ref[:, pl.ds(b * _RB, _RB)].reshape(_RB, 1)
        d2 = sqb + sq_row - 2.0 * p
        adjf = jnp.where(d2 <= _EPS2, 1.0, 0.0)
        m_ref[pl.ds(b * _RB, _RB), :] = adjf.astype(jnp.bfloat16)
        deg_ref[:, pl.ds(b * _RB, _RB)] = _row(jnp.sum(adjf, axis=1))
        return 0

    lax.fori_loop(0, _NB, build_blk, 0, unroll=False)

    core = deg_ref[...] >= float(_MINS)
    lab0 = jnp.where(core, iota_row, _SENT)

    def masked_min_pass(labm):
        labm_ref[...] = labm

        def cb_loop(cb, _):
            lsl = labm_ref[:, pl.ds(cb * _RB, _RB)]
            changed = jnp.any(lsl != prev_ref[:, pl.ds(cb * _RB, _RB)])

            def recompute():
                m = m_ref[:, pl.ds(cb * _RB, _RB)].astype(jnp.float32)
                t = m * lsl
                part_ref[cb, :] = jnp.min(t, axis=1)

            lax.cond(changed, recompute, lambda: None)
            return 0

        lax.fori_loop(0, _NB, cb_loop, 0, unroll=False)
        prev_ref[...] = labm
        return _row(jnp.min(part_ref[...], axis=0)) + _SENT

    def jump_gather():
        hm = hook_ref[...] - _SENT

        def blk(b, _):
            hb = hook_ref[:, pl.ds(b * _RB, _RB)].reshape(_RB, 1)
            p = (hb == iota_row).astype(jnp.float32)
            g = lax.dot_general(p, hm, (((1,), (1,)), ((), ())),
                                precision=lax.Precision.HIGHEST,
                                preferred_element_type=jnp.float32)
            vec_ref[:, pl.ds(b * _RB, _RB)] = g.reshape(1, _RB) + _SENT
            return 0
        lax.fori_loop(0, _NB, blk, 0, unroll=False)
        return vec_ref[...]

    prev_ref[...] = jnp.full((1, _N), 1.0, jnp.float32)

    def cond(carry):
        _, it, done = carry
        return jnp.logical_and(it < _MAXIT, jnp.logical_not(done))

    def body(carry):
        lab, it, _ = carry
        labm = jnp.where(core, lab, _SENT) - _SENT
        neigh = masked_min_pass(labm)
        hooked = jnp.where(core, jnp.minimum(lab, neigh), lab)
        hook_ref[...] = hooked
        done = jnp.all(hooked == lab)
        lab2 = lax.cond(
            it % 2 == 1,
            lambda: jnp.minimum(hook_ref[...], jump_gather()),
            lambda: hooked)
        return lab2, it + 1, done

    lab, _, _ = lax.while_loop(
        cond, body, (lab0, jnp.int32(0), jnp.bool_(False)))

    bord = _row(jnp.min(part_ref[...], axis=0)) + _SENT
    root = jnp.where(core, lab, bord)
    hook_ref[...] = root

    is_rootf = (jnp.logical_and(core, lab == iota_row)).astype(jnp.float32)

    c = is_rootf
    s = 1
    while s < _N:
        shifted = jnp.pad(c, ((0, 0), (s, 0)))[:, :_N]
        c = c + shifted
        s *= 2
    cid = c - 1.0
    deg_ref[...] = cid

    cidp = cid + 1.0

    def lab_blk(b, _):
        rb = hook_ref[:, pl.ds(b * _RB, _RB)].reshape(_RB, 1)
        p = (rb == iota_row).astype(jnp.float32)
        g = lax.dot_general(p, cidp, (((1,), (1,)), ((), ())),
                            precision=lax.Precision.HIGHEST,
                            preferred_element_type=jnp.float32)
        out_ref[:, pl.ds(b * _RB, _RB)] = (
            g.reshape(1, _RB) - 1.0).astype(jnp.int32)
        return 0

    lax.fori_loop(0, _NB, lab_blk, 0, unroll=False)


def kernel(X):
    out = pl.pallas_call(
        _dbscan_body,
        out_shape=jax.ShapeDtypeStruct((1, _N), jnp.int32),
        scratch_shapes=[
            pltpu.VMEM((_N, _N), jnp.bfloat16),
            pltpu.VMEM((1, _N), jnp.float32),
            pltpu.VMEM((1, _N), jnp.float32),
            pltpu.VMEM((1, _N), jnp.float32),
            pltpu.VMEM((1, _N), jnp.float32),
            pltpu.VMEM((1, _N), jnp.float32),
            pltpu.VMEM((_NB, _N), jnp.float32),
        ],
    )(X)
    return out.reshape(_N).astype(jnp.int64)


# speedup vs baseline: 1.0364x; 1.0364x over previous
import jax
import jax.numpy as jnp
from jax import lax
from jax.experimental import pallas as pl
from jax.experimental.pallas import tpu as pltpu

_EPS2 = 1.6 * 1.6
_MINS = 5
_N = 4096
_D = 8
_MAXIT = 64
_SENT = float(_N)
_RB = 256
_NB = _N // _RB


def _row(x):
    return x.reshape(1, -1)


def _dbscan_body(x_ref, out_ref, m_ref, deg_ref, vec_ref, hook_ref,
                 labm_ref, prev_ref, part_ref):
    x = x_ref[...]
    sq = jnp.sum(x * x, axis=1)
    sq_row = _row(sq)
    iota_row = lax.broadcasted_iota(jnp.int32, (1, _N), 1).astype(jnp.float32)

    hook_ref[...] = sq_row

    def build_blk(b, _):
        xb = x_ref[pl.ds(b * _RB, _RB), :]
        p = lax.dot_general(xb, x, (((1,), (1,)), ((), ())),
                            preferred_element_type=jnp.float32)
        sqb = hook_ref[:, pl.ds(b * _RB, _RB)].reshape(_RB, 1)
        d2 = sqb + sq_row - 2.0 * p
        adjf = jnp.where(d2 <= _EPS2, 1.0, 0.0)
        m_ref[pl.ds(b * _RB, _RB), :] = adjf.astype(jnp.bfloat16)
        deg_ref[:, pl.ds(b * _RB, _RB)] = _row(jnp.sum(adjf, axis=1))
        return 0

    lax.fori_loop(0, _NB, build_blk, 0, unroll=False)

    core = deg_ref[...] >= float(_MINS)
    lab0 = jnp.where(core, iota_row, _SENT)

    def masked_min_pass(labm):
        labm_ref[...] = labm

        def cb_loop(cb, _):
            lsl = labm_ref[:, pl.ds(cb * _RB, _RB)]
            changed = jnp.any(lsl != prev_ref[:, pl.ds(cb * _RB, _RB)])

            def recompute():
                m = m_ref[:, pl.ds(cb * _RB, _RB)].astype(jnp.float32)
                t = m * lsl
                part_ref[cb, :] = jnp.min(t, axis=1)

            lax.cond(changed, recompute, lambda: None)
            return 0

        lax.fori_loop(0, _NB, cb_loop, 0, unroll=False)
        prev_ref[...] = labm
        return _row(jnp.min(part_ref[...], axis=0)) + _SENT

    def jump_gather():
        h = hook_ref[...]

        def blk(b, _):
            hb = hook_ref[:, pl.ds(b * _RB, _RB)].reshape(_RB, 1)
            t = jnp.where(hb == iota_row, h, _SENT)
            vec_ref[:, pl.ds(b * _RB, _RB)] = _row(jnp.min(t, axis=1))
            return 0
        lax.fori_loop(0, _NB, blk, 0, unroll=False)
        return vec_ref[...]

    prev_ref[...] = jnp.full((1, _N), 1.0, jnp.float32)

    def cond(carry):
        _, it, done = carry
        return jnp.logical_and(it < _MAXIT, jnp.logical_not(done))

    def body(carry):
        lab, it, _ = carry
        labm = jnp.where(core, lab, _SENT) - _SENT
        neigh = masked_min_pass(labm)
        hooked = jnp.where(core, jnp.minimum(lab, neigh), lab)
        hook_ref[...] = hooked
        done = jnp.all(hooked == lab)
        lab2 = lax.cond(
            it % 2 == 1,
            lambda: jnp.minimum(hook_ref[...], jump_gather()),
            lambda: hooked)
        return lab2, it + 1, done

    lab, _, _ = lax.while_loop(
        cond, body, (lab0, jnp.int32(0), jnp.bool_(False)))

    bord = _row(jnp.min(part_ref[...], axis=0)) + _SENT
    root = jnp.where(core, lab, bord)
    hook_ref[...] = root

    is_rootf = (jnp.logical_and(core, lab == iota_row)).astype(jnp.float32)

    c = is_rootf
    s = 1
    while s < _N:
        shifted = jnp.pad(c, ((0, 0), (s, 0)))[:, :_N]
        c = c + shifted
        s *= 2
    cid = c - 1.0
    deg_ref[...] = cid

    def lab_blk(b, _):
        rb = hook_ref[:, pl.ds(b * _RB, _RB)].reshape(_RB, 1)
        t = jnp.where(rb == iota_row, deg_ref[...], _SENT)
        out_ref[:, pl.ds(b * _RB, _RB)] = _row(
            jnp.min(t, axis=1)).astype(jnp.int32)
        return 0

    lax.fori_loop(0, _NB, lab_blk, 0, unroll=False)
    noise = hook_ref[...] == _SENT
    out_ref[...] = jnp.where(noise, -1, out_ref[...])


def kernel(X):
    out = pl.pallas_call(
        _dbscan_body,
        out_shape=jax.ShapeDtypeStruct((1, _N), jnp.int32),
        scratch_shapes=[
            pltpu.VMEM((_N, _N), jnp.bfloat16),
            pltpu.VMEM((1, _N), jnp.float32),
            pltpu.VMEM((1, _N), jnp.float32),
            pltpu.VMEM((1, _N), jnp.float32),
            pltpu.VMEM((1, _N), jnp.float32),
            pltpu.VMEM((1, _N), jnp.float32),
            pltpu.VMEM((_NB, _N), jnp.float32),
        ],
    )(X)
    return out.reshape(_N).astype(jnp.int64)


# speedup vs baseline: 2.0749x; 2.0021x over previous
import jax
import jax.numpy as jnp
from jax import lax
from jax.experimental import pallas as pl
from jax.experimental.pallas import tpu as pltpu

_EPS2 = 1.6 * 1.6
_MINS = 5
_N = 4096
_D = 8
_MAXIT = 64
_SENT = float(_N)
_RB = 256
_NB = _N // _RB


def _row(x):
    return x.reshape(1, -1)


def _dbscan_body(x_ref, out_ref, m_ref, deg_ref, vec_ref, hook_ref,
                 labm_ref, prev_ref, part_ref):
    x = x_ref[...]
    sq = jnp.sum(x * x, axis=1)
    sq_row = _row(sq)
    iota_row = lax.broadcasted_iota(jnp.int32, (1, _N), 1).astype(jnp.float32)

    hook_ref[...] = sq_row

    def build_blk(b, _):
        xb = x_ref[pl.ds(b * _RB, _RB), :]
        p = lax.dot_general(xb, x, (((1,), (1,)), ((), ())),
                            preferred_element_type=jnp.float32)
        sqb = hook_ref[:, pl.ds(b * _RB, _RB)].reshape(_RB, 1)
        d2 = sqb + sq_row - 2.0 * p
        adjf = jnp.where(d2 <= _EPS2, 1.0, 0.0)
        m_ref[pl.ds(b * _RB, _RB), :] = adjf.astype(jnp.bfloat16)
        deg_ref[:, pl.ds(b * _RB, _RB)] = _row(jnp.sum(adjf, axis=1))
        return 0

    lax.fori_loop(0, _NB, build_blk, 0, unroll=False)

    core = deg_ref[...] >= float(_MINS)
    lab0 = jnp.where(core, iota_row, _SENT)

    def masked_min_pass(labm):
        labm_ref[...] = labm

        def cb_loop(cb, _):
            lsl = labm_ref[:, pl.ds(cb * _RB, _RB)]
            changed = jnp.any(lsl != prev_ref[:, pl.ds(cb * _RB, _RB)])

            def recompute():
                m = m_ref[pl.ds(cb * _RB, _RB), :].astype(jnp.float32)
                t = m * lsl.reshape(_RB, 1)
                part_ref[cb, :] = jnp.min(t, axis=0)

            lax.cond(changed, recompute, lambda: None)
            return 0

        lax.fori_loop(0, _NB, cb_loop, 0, unroll=False)
        prev_ref[...] = labm
        return _row(jnp.min(part_ref[...], axis=0)) + _SENT

    def jump_gather():
        h = hook_ref[...]

        def blk(b, _):
            hb = hook_ref[:, pl.ds(b * _RB, _RB)].reshape(_RB, 1)
            t = jnp.where(hb == iota_row, h, _SENT)
            vec_ref[:, pl.ds(b * _RB, _RB)] = _row(jnp.min(t, axis=1))
            return 0
        lax.fori_loop(0, _NB, blk, 0, unroll=False)
        return vec_ref[...]

    prev_ref[...] = jnp.full((1, _N), 1.0, jnp.float32)

    def cond(carry):
        _, it, done = carry
        return jnp.logical_and(it < _MAXIT, jnp.logical_not(done))

    def body(carry):
        lab, it, _ = carry
        labm = jnp.where(core, lab, _SENT) - _SENT
        neigh = masked_min_pass(labm)
        hooked = jnp.where(core, jnp.minimum(lab, neigh), lab)
        hook_ref[...] = hooked
        done = jnp.all(hooked == lab)
        lab2 = lax.cond(
            it % 2 == 1,
            lambda: jnp.minimum(hook_ref[...], jump_gather()),
            lambda: hooked)
        return lab2, it + 1, done

    lab, _, _ = lax.while_loop(
        cond, body, (lab0, jnp.int32(0), jnp.bool_(False)))

    bord = _row(jnp.min(part_ref[...], axis=0)) + _SENT
    root = jnp.where(core, lab, bord)
    hook_ref[...] = root

    is_rootf = (jnp.logical_and(core, lab == iota_row)).astype(jnp.float32)

    c = is_rootf
    s = 1
    while s < _N:
        shifted = jnp.pad(c, ((0, 0), (s, 0)))[:, :_N]
        c = c + shifted
        s *= 2
    cid = c - 1.0
    deg_ref[...] = cid

    def lab_blk(b, _):
        rb = hook_ref[:, pl.ds(b * _RB, _RB)].reshape(_RB, 1)
        t = jnp.where(rb == iota_row, deg_ref[...], _SENT)
        out_ref[:, pl.ds(b * _RB, _RB)] = _row(
            jnp.min(t, axis=1)).astype(jnp.int32)
        return 0

    lax.fori_loop(0, _NB, lab_blk, 0, unroll=False)
    noise = hook_ref[...] == _SENT
    out_ref[...] = jnp.where(noise, -1, out_ref[...])


def kernel(X):
    out = pl.pallas_call(
        _dbscan_body,
        out_shape=jax.ShapeDtypeStruct((1, _N), jnp.int32),
        scratch_shapes=[
            pltpu.VMEM((_N, _N), jnp.bfloat16),
            pltpu.VMEM((1, _N), jnp.float32),
            pltpu.VMEM((1, _N), jnp.float32),
            pltpu.VMEM((1, _N), jnp.float32),
            pltpu.VMEM((1, _N), jnp.float32),
            pltpu.VMEM((1, _N), jnp.float32),
            pltpu.VMEM((_NB, _N), jnp.float32),
        ],
    )(X)
    return out.reshape(_N).astype(jnp.int64)


# speedup vs baseline: 2.3332x; 1.1245x over previous
import jax
import jax.numpy as jnp
from jax import lax
from jax.experimental import pallas as pl
from jax.experimental.pallas import tpu as pltpu

_EPS2 = 1.6 * 1.6
_MINS = 5
_N = 4096
_D = 8
_MAXIT = 96
_SENT = float(_N)
_RB = 256
_NB = _N // _RB


def _row(x):
    return x.reshape(1, -1)


def _dbscan_body(x_ref, out_ref, m_ref, deg_ref, vec_ref, hook_ref,
                 labm_ref, prev_ref, part_ref):
    x = x_ref[...]
    sq = jnp.sum(x * x, axis=1)
    sq_row = _row(sq)
    iota_row = lax.broadcasted_iota(jnp.int32, (1, _N), 1).astype(jnp.float32)

    hook_ref[...] = sq_row

    def build_blk(b, _):
        xb = x_ref[pl.ds(b * _RB, _RB), :]
        p = lax.dot_general(xb, x, (((1,), (1,)), ((), ())),
                            preferred_element_type=jnp.float32)
        sqb = hook_ref[:, pl.ds(b * _RB, _RB)].reshape(_RB, 1)
        d2 = sqb + sq_row - 2.0 * p
        adjf = jnp.where(d2 <= _EPS2, 1.0, 0.0)
        m_ref[pl.ds(b * _RB, _RB), :] = adjf.astype(jnp.bfloat16)
        deg_ref[:, pl.ds(b * _RB, _RB)] = _row(jnp.sum(adjf, axis=1))
        return 0

    lax.fori_loop(0, _NB, build_blk, 0, unroll=2)

    core = deg_ref[...] >= float(_MINS)
    lab0 = jnp.where(core, iota_row, _SENT)

    def masked_min_pass(labm):
        labm_ref[...] = labm

        def cb_loop(cb, _):
            lsl = labm_ref[:, pl.ds(cb * _RB, _RB)]
            changed = jnp.any(lsl != prev_ref[:, pl.ds(cb * _RB, _RB)])

            def recompute():
                m = m_ref[pl.ds(cb * _RB, _RB), :].astype(jnp.float32)
                t = m * lsl.reshape(_RB, 1)
                part_ref[cb, :] = jnp.min(t, axis=0)

            lax.cond(changed, recompute, lambda: None)
            return 0

        lax.fori_loop(0, _NB, cb_loop, 0, unroll=False)
        prev_ref[...] = labm
        return _row(jnp.min(part_ref[...], axis=0)) + _SENT

    def jump_gather():
        h = hook_ref[...]

        def blk(b, _):
            hb = hook_ref[:, pl.ds(b * _RB, _RB)].reshape(_RB, 1)
            t = jnp.where(hb == iota_row, h, _SENT)
            vec_ref[:, pl.ds(b * _RB, _RB)] = _row(jnp.min(t, axis=1))
            return 0
        lax.fori_loop(0, _NB, blk, 0, unroll=False)
        return vec_ref[...]

    prev_ref[...] = jnp.full((1, _N), 1.0, jnp.float32)

    def cond(carry):
        _, it, done = carry
        return jnp.logical_and(it < _MAXIT, jnp.logical_not(done))

    def body(carry):
        lab, it, _ = carry
        labm = jnp.where(core, lab, _SENT) - _SENT
        neigh = masked_min_pass(labm)
        hooked = jnp.where(core, jnp.minimum(lab, neigh), lab)
        hook_ref[...] = hooked
        done = jnp.all(hooked == lab)
        lab2 = lax.cond(
            it % 4 == 3,
            lambda: jnp.minimum(hook_ref[...], jump_gather()),
            lambda: hooked)
        return lab2, it + 1, done

    lab, _, _ = lax.while_loop(
        cond, body, (lab0, jnp.int32(0), jnp.bool_(False)))

    bord = _row(jnp.min(part_ref[...], axis=0)) + _SENT
    root = jnp.where(core, lab, bord)
    hook_ref[...] = root

    is_rootf = (jnp.logical_and(core, lab == iota_row)).astype(jnp.float32)

    c = is_rootf
    s = 1
    while s < _N:
        shifted = jnp.pad(c, ((0, 0), (s, 0)))[:, :_N]
        c = c + shifted
        s *= 2
    cid = c - 1.0
    deg_ref[...] = cid

    def lab_blk(b, _):
        rb = hook_ref[:, pl.ds(b * _RB, _RB)].reshape(_RB, 1)
        t = jnp.where(rb == iota_row, deg_ref[...], _SENT)
        out_ref[:, pl.ds(b * _RB, _RB)] = _row(
            jnp.min(t, axis=1)).astype(jnp.int32)
        return 0

    lax.fori_loop(0, _NB, lab_blk, 0, unroll=False)
    noise = hook_ref[...] == _SENT
    out_ref[...] = jnp.where(noise, -1, out_ref[...])


def kernel(X):
    out = pl.pallas_call(
        _dbscan_body,
        out_shape=jax.ShapeDtypeStruct((1, _N), jnp.int32),
        scratch_shapes=[
            pltpu.VMEM((_N, _N), jnp.bfloat16),
            pltpu.VMEM((1, _N), jnp.float32),
            pltpu.VMEM((1, _N), jnp.float32),
            pltpu.VMEM((1, _N), jnp.float32),
            pltpu.VMEM((1, _N), jnp.float32),
            pltpu.VMEM((1, _N), jnp.float32),
            pltpu.VMEM((_NB, _N), jnp.float32),
        ],
    )(X)
    return out.reshape(_N).astype(jnp.int64)


# speedup vs baseline: 2.4108x; 1.0333x over previous
import jax
import jax.numpy as jnp
from jax import lax
from jax.experimental import pallas as pl
from jax.experimental.pallas import tpu as pltpu

_EPS2 = 1.6 * 1.6
_MINS = 5
_N = 4096
_D = 8
_MAXIT = 96
_SENT = float(_N)
_RB = 256
_NB = _N // _RB


def _row(x):
    return x.reshape(1, -1)


def _dbscan_body(x_ref, out_ref, m_ref, deg_ref, vec_ref, hook_ref,
                 labm_ref, prev_ref, part_ref):
    x = x_ref[...]
    sq = jnp.sum(x * x, axis=1)
    sq_row = _row(sq)
    iota_row = lax.broadcasted_iota(jnp.int32, (1, _N), 1).astype(jnp.float32)

    hook_ref[...] = sq_row

    def build_blk(b, _):
        xb = x_ref[pl.ds(b * _RB, _RB), :]
        p = lax.dot_general(xb, x, (((1,), (1,)), ((), ())),
                            preferred_element_type=jnp.float32)
        sqb = hook_ref[:, pl.ds(b * _RB, _RB)].reshape(_RB, 1)
        d2 = sqb + sq_row - 2.0 * p
        adjf = jnp.where(d2 <= _EPS2, 1.0, 0.0)
        m_ref[pl.ds(b * _RB, _RB), :] = adjf.astype(jnp.bfloat16)
        deg_ref[:, pl.ds(b * _RB, _RB)] = _row(jnp.sum(adjf, axis=1))
        return 0

    lax.fori_loop(0, _NB, build_blk, 0, unroll=2)

    core = deg_ref[...] >= float(_MINS)
    lab0 = jnp.where(core, iota_row, _SENT)

    def masked_min_pass(labm):
        labm_ref[...] = labm

        def cb_loop(cb, _):
            lsl = labm_ref[:, pl.ds(cb * _RB, _RB)]
            changed = jnp.any(lsl != prev_ref[:, pl.ds(cb * _RB, _RB)])

            def recompute():
                m = m_ref[pl.ds(cb * _RB, _RB), :].astype(jnp.float32)
                t = m * lsl.reshape(_RB, 1)
                part_ref[cb, :] = jnp.min(t, axis=0)

            lax.cond(changed, recompute, lambda: None)
            return 0

        lax.fori_loop(0, _NB, cb_loop, 0, unroll=2)
        prev_ref[...] = labm
        return _row(jnp.min(part_ref[...], axis=0)) + _SENT

    def jump_gather():
        h = hook_ref[...]

        def blk(b, _):
            hb = hook_ref[:, pl.ds(b * _RB, _RB)].reshape(_RB, 1)
            t = jnp.where(hb == iota_row, h, _SENT)
            vec_ref[:, pl.ds(b * _RB, _RB)] = _row(jnp.min(t, axis=1))
            return 0
        lax.fori_loop(0, _NB, blk, 0, unroll=2)
        return vec_ref[...]

    prev_ref[...] = jnp.full((1, _N), 1.0, jnp.float32)

    def cond(carry):
        _, it, done = carry
        return jnp.logical_and(it < _MAXIT, jnp.logical_not(done))

    def body(carry):
        lab, it, _ = carry
        labm = jnp.where(core, lab, _SENT) - _SENT
        neigh = masked_min_pass(labm)
        hooked = jnp.where(core, jnp.minimum(lab, neigh), lab)
        hook_ref[...] = hooked
        done = jnp.all(hooked == lab)
        lab2 = lax.cond(
            it % 4 == 3,
            lambda: jnp.minimum(hook_ref[...], jump_gather()),
            lambda: hooked)
        return lab2, it + 1, done

    lab, _, _ = lax.while_loop(
        cond, body, (lab0, jnp.int32(0), jnp.bool_(False)))

    bord = _row(jnp.min(part_ref[...], axis=0)) + _SENT
    root = jnp.where(core, lab, bord)
    hook_ref[...] = root

    is_rootf = (jnp.logical_and(core, lab == iota_row)).astype(jnp.float32)

    c = is_rootf
    s = 1
    while s < _N:
        shifted = jnp.pad(c, ((0, 0), (s, 0)))[:, :_N]
        c = c + shifted
        s *= 2
    cid = c - 1.0
    deg_ref[...] = cid

    def lab_blk(b, _):
        rb = hook_ref[:, pl.ds(b * _RB, _RB)].reshape(_RB, 1)
        t = jnp.where(rb == iota_row, deg_ref[...], _SENT)
        out_ref[:, pl.ds(b * _RB, _RB)] = _row(
            jnp.min(t, axis=1)).astype(jnp.int32)
        return 0

    lax.fori_loop(0, _NB, lab_blk, 0, unroll=2)
    noise = hook_ref[...] == _SENT
    out_ref[...] = jnp.where(noise, -1, out_ref[...])


def kernel(X):
    out = pl.pallas_call(
        _dbscan_body,
        out_shape=jax.ShapeDtypeStruct((1, _N), jnp.int32),
        scratch_shapes=[
            pltpu.VMEM((_N, _N), jnp.bfloat16),
            pltpu.VMEM((1, _N), jnp.float32),
            pltpu.VMEM((1, _N), jnp.float32),
            pltpu.VMEM((1, _N), jnp.float32),
            pltpu.VMEM((1, _N), jnp.float32),
            pltpu.VMEM((1, _N), jnp.float32),
            pltpu.VMEM((_NB, _N), jnp.float32),
        ],
    )(X)
    return out.reshape(_N).astype(jnp.int64)


# speedup vs baseline: 2.7303x; 1.1325x over previous
import jax
import jax.numpy as jnp
from jax import lax
from jax.experimental import pallas as pl
from jax.experimental.pallas import tpu as pltpu

_EPS2 = 1.6 * 1.6
_MINS = 5
_N = 4096
_D = 8
_MAXIT = 96
_SENT = float(_N)
_RB = 512
_NB = _N // _RB


def _row(x):
    return x.reshape(1, -1)


def _dbscan_body(x_ref, out_ref, m_ref, deg_ref, vec_ref, hook_ref,
                 labm_ref, prev_ref, part_ref):
    x = x_ref[...]
    sq = jnp.sum(x * x, axis=1)
    sq_row = _row(sq)
    iota_row = lax.broadcasted_iota(jnp.int32, (1, _N), 1).astype(jnp.float32)

    hook_ref[...] = sq_row

    def build_blk(b, _):
        xb = x_ref[pl.ds(b * _RB, _RB), :]
        p = lax.dot_general(xb, x, (((1,), (1,)), ((), ())),
                            preferred_element_type=jnp.float32)
        sqb = hook_ref[:, pl.ds(b * _RB, _RB)].reshape(_RB, 1)
        d2 = sqb + sq_row - 2.0 * p
        adjf = jnp.where(d2 <= _EPS2, 1.0, 0.0)
        m_ref[pl.ds(b * _RB, _RB), :] = adjf.astype(jnp.bfloat16)
        deg_ref[:, pl.ds(b * _RB, _RB)] = _row(jnp.sum(adjf, axis=1))
        return 0

    lax.fori_loop(0, _NB, build_blk, 0, unroll=2)

    core = deg_ref[...] >= float(_MINS)
    lab0 = jnp.where(core, iota_row, _SENT)

    def masked_min_pass(labm):
        labm_ref[...] = labm

        def cb_loop(cb, _):
            lsl = labm_ref[:, pl.ds(cb * _RB, _RB)]
            changed = jnp.any(lsl != prev_ref[:, pl.ds(cb * _RB, _RB)])

            def recompute():
                m = m_ref[pl.ds(cb * _RB, _RB), :].astype(jnp.float32)
                t = m * lsl.reshape(_RB, 1)
                part_ref[cb, :] = jnp.min(t, axis=0)

            lax.cond(changed, recompute, lambda: None)
            return 0

        lax.fori_loop(0, _NB, cb_loop, 0, unroll=2)
        prev_ref[...] = labm
        return _row(jnp.min(part_ref[...], axis=0)) + _SENT

    def jump_gather():
        h = hook_ref[...]

        def blk(b, _):
            hb = hook_ref[:, pl.ds(b * _RB, _RB)].reshape(_RB, 1)
            t = jnp.where(hb == iota_row, h, _SENT)
            vec_ref[:, pl.ds(b * _RB, _RB)] = _row(jnp.min(t, axis=1))
            return 0
        lax.fori_loop(0, _NB, blk, 0, unroll=2)
        return vec_ref[...]

    prev_ref[...] = jnp.full((1, _N), 1.0, jnp.float32)

    def cond(carry):
        _, it, done = carry
        return jnp.logical_and(it < _MAXIT, jnp.logical_not(done))

    def body(carry):
        lab, it, _ = carry
        labm = jnp.where(core, lab, _SENT) - _SENT
        neigh = masked_min_pass(labm)
        hooked = jnp.where(core, jnp.minimum(lab, neigh), lab)
        hook_ref[...] = hooked
        done = jnp.all(hooked == lab)
        lab2 = lax.cond(
            it % 4 == 3,
            lambda: jnp.minimum(hook_ref[...], jump_gather()),
            lambda: hooked)
        return lab2, it + 1, done

    lab, _, _ = lax.while_loop(
        cond, body, (lab0, jnp.int32(0), jnp.bool_(False)))

    bord = _row(jnp.min(part_ref[...], axis=0)) + _SENT
    root = jnp.where(core, lab, bord)
    hook_ref[...] = root

    is_rootf = (jnp.logical_and(core, lab == iota_row)).astype(jnp.float32)

    c = is_rootf
    s = 1
    while s < _N:
        shifted = jnp.pad(c, ((0, 0), (s, 0)))[:, :_N]
        c = c + shifted
        s *= 2
    cid = c - 1.0
    deg_ref[...] = cid

    def lab_blk(b, _):
        rb = hook_ref[:, pl.ds(b * _RB, _RB)].reshape(_RB, 1)
        t = jnp.where(rb == iota_row, deg_ref[...], _SENT)
        out_ref[:, pl.ds(b * _RB, _RB)] = _row(
            jnp.min(t, axis=1)).astype(jnp.int32)
        return 0

    lax.fori_loop(0, _NB, lab_blk, 0, unroll=2)
    noise = hook_ref[...] == _SENT
    out_ref[...] = jnp.where(noise, -1, out_ref[...])


def kernel(X):
    out = pl.pallas_call(
        _dbscan_body,
        out_shape=jax.ShapeDtypeStruct((1, _N), jnp.int32),
        scratch_shapes=[
            pltpu.VMEM((_N, _N), jnp.bfloat16),
            pltpu.VMEM((1, _N), jnp.float32),
            pltpu.VMEM((1, _N), jnp.float32),
            pltpu.VMEM((1, _N), jnp.float32),
            pltpu.VMEM((1, _N), jnp.float32),
            pltpu.VMEM((1, _N), jnp.float32),
            pltpu.VMEM((_NB, _N), jnp.float32),
        ],
    )(X)
    return out.reshape(_N).astype(jnp.int64)
